# baseline (speedup 1.0000x reference)
import functools

import jax
import jax.numpy as jnp
from jax import lax
from jax.experimental import pallas as pl
from jax.experimental.pallas import tpu as pltpu

_EPS = 1e-5


def _pick_block(n):
    for blk in (5000, 4096, 4000, 2048, 2000, 1024, 1000):
        if n % blk == 0:
            return blk
    return n


def _mlp_kernel(x_ref, W1_ref, b1_ref, g1_ref, be1_ref, W2_ref, b2_ref,
                g2_ref, be2_ref, out_ref,
                xc_ref, s1_ref, q1_ref, s2_ref, q2_ref, W1s_ref, bias1_ref,
                *, nb, blk, inv_n):
    t = pl.program_id(0)
    phase = t // nb
    i = lax.rem(t, nb)

    @pl.when(t == 0)
    def _init1():
        s1_ref[...] = jnp.zeros_like(s1_ref)
        q1_ref[...] = jnp.zeros_like(q1_ref)

    @pl.when(phase == 0)
    def _p0():
        xb = x_ref[...]
        s1_ref[...] += jnp.sum(xb, axis=0, keepdims=True)
        q1_ref[...] += jnp.sum(xb * xb, axis=0, keepdims=True)
        xc_ref[pl.ds(i * blk, blk), :] = xb.astype(jnp.bfloat16)

    @pl.when(t == nb)
    def _fold_bn1():
        m1 = s1_ref[...] * inv_n
        v1 = q1_ref[...] * inv_n - m1 * m1
        a1 = g1_ref[...] * lax.rsqrt(v1 + _EPS)
        c1 = be1_ref[...] - m1 * a1
        W1s_ref[...] = (W1_ref[...] * a1).astype(jnp.bfloat16)
        bias1_ref[...] = lax.dot_general(
            c1, W1_ref[...], (((1,), (1,)), ((), ())),
            preferred_element_type=jnp.float32,
            precision=lax.Precision.HIGHEST) + b1_ref[...]
        s2_ref[...] = jnp.zeros_like(s2_ref)
        q2_ref[...] = jnp.zeros_like(q2_ref)

    def hidden():
        xcb = xc_ref[pl.ds(i * blk, blk), :]
        z = lax.dot_general(xcb, W1s_ref[...], (((1,), (1,)), ((), ())),
                            preferred_element_type=jnp.float32)
        return jnp.maximum(z + bias1_ref[...], 0.0)

    @pl.when(phase == 1)
    def _p1():
        h = hidden()
        s2_ref[...] += jnp.sum(h, axis=0, keepdims=True)
        q2_ref[...] += jnp.sum(h * h, axis=0, keepdims=True)

    @pl.when(phase == 2)
    def _p2():
        m2 = s2_ref[...] * inv_n
        v2 = q2_ref[...] * inv_n - m2 * m2
        a2 = g2_ref[...] * lax.rsqrt(v2 + _EPS)
        c2 = be2_ref[...] - m2 * a2
        W2s = W2_ref[...] * a2
        bias2 = lax.dot_general(
            c2, W2_ref[...], (((1,), (1,)), ((), ())),
            preferred_element_type=jnp.float32,
            precision=lax.Precision.HIGHEST) + b2_ref[...]
        out = lax.dot_general(hidden(), W2s, (((1,), (1,)), ((), ())),
                              preferred_element_type=jnp.float32,
                              precision=lax.Precision.HIGHEST)
        out_ref[...] = out + bias2


def kernel(causal, gamma1, beta1, W1, b1, gamma2, beta2, W2, b2):
    n, d = causal.shape
    d_out = W2.shape[0]
    blk = _pick_block(n)
    nb = n // blk

    row = lambda v: v.reshape(1, -1)

    def full(shape):
        return pl.BlockSpec(shape, lambda t: (0,) * len(shape))

    x_spec = pl.BlockSpec((blk, d), lambda t: (jnp.minimum(t, nb - 1), 0))
    out_spec = pl.BlockSpec(
        (blk, d_out),
        lambda t: (jnp.where(t >= 2 * nb, lax.rem(t, nb), 0), 0))

    fn = pl.pallas_call(
        functools.partial(_mlp_kernel, nb=nb, blk=blk, inv_n=1.0 / n),
        grid=(3 * nb,),
        in_specs=[
            x_spec,
            full((d, d)),
            full((1, d)),
            full((1, d)),
            full((1, d)),
            full((d_out, d)),
            full((1, d_out)),
            full((1, d)),
            full((1, d)),
        ],
        out_specs=out_spec,
        out_shape=jax.ShapeDtypeStruct((n, d_out), jnp.float32),
        scratch_shapes=[
            pltpu.VMEM((n, d), jnp.bfloat16),
            pltpu.VMEM((1, d), jnp.float32),
            pltpu.VMEM((1, d), jnp.float32),
            pltpu.VMEM((1, d), jnp.float32),
            pltpu.VMEM((1, d), jnp.float32),
            pltpu.VMEM((d, d), jnp.bfloat16),
            pltpu.VMEM((1, d), jnp.float32),
        ],
        compiler_params=pltpu.CompilerParams(
            dimension_semantics=("arbitrary",)),
    )
    return fn(causal, W1, row(b1), row(gamma1), row(beta1),
              W2, row(b2), row(gamma2), row(beta2))


# speedup vs baseline: 1.3184x; 1.3184x over previous
import functools

import jax
import jax.numpy as jnp
from jax import lax
from jax.experimental import pallas as pl
from jax.experimental.pallas import tpu as pltpu

_EPS = 1e-5


def _pick_block(n):
    for blk in (4000, 2048, 2000, 1024, 1000, 512, 500):
        if n % blk == 0:
            return blk
    return n


def _mlp_kernel(x_ref, W1_ref, b1_ref, g1_ref, be1_ref, W2_ref, b2_ref,
                g2_ref, be2_ref, out_ref,
                xc_ref, s1_ref, q1_ref, s2_ref, q2_ref, W1s_ref, bias1_ref,
                *, nb, blk, inv_n):
    t = pl.program_id(0)
    phase = t // nb
    i = lax.rem(t, nb)

    @pl.when(t == 0)
    def _init1():
        s1_ref[...] = jnp.zeros_like(s1_ref)
        q1_ref[...] = jnp.zeros_like(q1_ref)

    @pl.when(phase == 0)
    def _p0():
        xb = x_ref[...]
        s1_ref[...] += jnp.sum(xb, axis=0, keepdims=True)
        q1_ref[...] += jnp.sum(xb * xb, axis=0, keepdims=True)
        xc_ref[pl.ds(i * blk, blk), :] = xb.astype(jnp.bfloat16)

    @pl.when(t == nb)
    def _fold_bn1():
        m1 = s1_ref[...] * inv_n
        v1 = q1_ref[...] * inv_n - m1 * m1
        a1 = g1_ref[...] * lax.rsqrt(v1 + _EPS)
        c1 = be1_ref[...] - m1 * a1
        W1s_ref[...] = (W1_ref[...] * a1).astype(jnp.bfloat16)
        bias1_ref[...] = lax.dot_general(
            c1, W1_ref[...], (((1,), (1,)), ((), ())),
            preferred_element_type=jnp.float32,
            precision=lax.Precision.HIGHEST) + b1_ref[...]
        s2_ref[...] = jnp.zeros_like(s2_ref)
        q2_ref[...] = jnp.zeros_like(q2_ref)

    def hidden():
        xcb = xc_ref[pl.ds(i * blk, blk), :]
        z = lax.dot_general(xcb, W1s_ref[...], (((1,), (1,)), ((), ())),
                            preferred_element_type=jnp.float32)
        return jnp.maximum(z + bias1_ref[...], 0.0)

    @pl.when(phase == 1)
    def _p1():
        h = hidden()
        s2_ref[...] += jnp.sum(h, axis=0, keepdims=True)
        q2_ref[...] += jnp.sum(h * h, axis=0, keepdims=True)

    @pl.when(phase == 2)
    def _p2():
        m2 = s2_ref[...] * inv_n
        v2 = q2_ref[...] * inv_n - m2 * m2
        a2 = g2_ref[...] * lax.rsqrt(v2 + _EPS)
        c2 = be2_ref[...] - m2 * a2
        W2s = W2_ref[...] * a2
        bias2 = lax.dot_general(
            c2, W2_ref[...], (((1,), (1,)), ((), ())),
            preferred_element_type=jnp.float32,
            precision=lax.Precision.HIGHEST) + b2_ref[...]
        out = lax.dot_general(hidden(), W2s, (((1,), (1,)), ((), ())),
                              preferred_element_type=jnp.float32,
                              precision=lax.Precision.HIGHEST)
        out_ref[...] = out + bias2


def kernel(causal, gamma1, beta1, W1, b1, gamma2, beta2, W2, b2):
    n, d = causal.shape
    d_out = W2.shape[0]
    blk = _pick_block(n)
    nb = n // blk

    row = lambda v: v.reshape(1, -1)

    def full(shape):
        return pl.BlockSpec(shape, lambda t: (0,) * len(shape))

    x_spec = pl.BlockSpec((blk, d), lambda t: (jnp.minimum(t, nb - 1), 0))
    out_spec = pl.BlockSpec(
        (blk, d_out),
        lambda t: (jnp.where(t >= 2 * nb, lax.rem(t, nb), 0), 0))

    fn = pl.pallas_call(
        functools.partial(_mlp_kernel, nb=nb, blk=blk, inv_n=1.0 / n),
        grid=(3 * nb,),
        in_specs=[
            x_spec,
            full((d, d)),
            full((1, d)),
            full((1, d)),
            full((1, d)),
            full((d_out, d)),
            full((1, d_out)),
            full((1, d)),
            full((1, d)),
        ],
        out_specs=out_spec,
        out_shape=jax.ShapeDtypeStruct((n, d_out), jnp.float32),
        scratch_shapes=[
            pltpu.VMEM((n, d), jnp.bfloat16),
            pltpu.VMEM((1, d), jnp.float32),
            pltpu.VMEM((1, d), jnp.float32),
            pltpu.VMEM((1, d), jnp.float32),
            pltpu.VMEM((1, d), jnp.float32),
            pltpu.VMEM((d, d), jnp.bfloat16),
            pltpu.VMEM((1, d), jnp.float32),
        ],
        compiler_params=pltpu.CompilerParams(
            dimension_semantics=("arbitrary",)),
    )
    return fn(causal, W1, row(b1), row(gamma1), row(beta1),
              W2, row(b2), row(gamma2), row(beta2))


# speedup vs baseline: 1.6238x; 1.2317x over previous
import functools

import jax
import jax.numpy as jnp
from jax import lax
from jax.experimental import pallas as pl
from jax.experimental.pallas import tpu as pltpu

_EPS = 1e-5


def _pick_block(n):
    for blk in (4000, 2048, 2000, 1024, 1000, 512, 496, 256):
        if n % blk == 0:
            return blk
    return n


def _mlp_kernel(x_hbm, W1_ref, b1_ref, g1_ref, be1_ref, W2_ref, b2_ref,
                g2_ref, be2_ref, out_ref, xc_ref, xbuf_ref, sem,
                *, nb, blk, inv_n):
    d = W1_ref.shape[0]

    def copy_in(slot, i):
        return pltpu.make_async_copy(
            x_hbm.at[pl.ds(i * blk, blk), :], xbuf_ref.at[slot],
            sem.at[slot])

    copy_in(0, 0).start()

    def p0(i, carry):
        s1, q1 = carry
        slot = lax.rem(i, 2)

        @pl.when(i + 1 < nb)
        def _prefetch():
            copy_in(lax.rem(i + 1, 2), i + 1).start()

        copy_in(slot, i).wait()
        xb = xbuf_ref[slot]
        s1 = s1 + jnp.sum(xb, axis=0, keepdims=True)
        q1 = q1 + jnp.sum(xb * xb, axis=0, keepdims=True)
        xc_ref[pl.ds(i * blk, blk), :] = xb.astype(jnp.bfloat16)
        return s1, q1

    zrow = jnp.zeros((1, d), jnp.float32)
    s1, q1 = lax.fori_loop(0, nb, p0, (zrow, zrow))

    def bn_affine(s, q, g_ref, be_ref):
        mean = s * inv_n
        var = q * inv_n - mean * mean
        a = g_ref[...] * lax.rsqrt(var + _EPS)
        c = be_ref[...] - mean * a
        return a, c

    a1, c1 = bn_affine(s1, q1, g1_ref, be1_ref)
    W1s = (W1_ref[...] * a1).astype(jnp.bfloat16)
    bias1 = lax.dot_general(c1, W1_ref[...], (((1,), (1,)), ((), ())),
                            preferred_element_type=jnp.float32,
                            precision=lax.Precision.HIGHEST) + b1_ref[...]

    def hblock(i):
        xcb = xc_ref[pl.ds(i * blk, blk), :]
        z = lax.dot_general(xcb, W1s, (((1,), (1,)), ((), ())),
                            preferred_element_type=jnp.float32)
        return jnp.maximum(z + bias1, 0.0)

    def p1(i, carry):
        s2, q2 = carry
        h = hblock(i)
        s2 = s2 + jnp.sum(h, axis=0, keepdims=True)
        q2 = q2 + jnp.sum(h * h, axis=0, keepdims=True)
        return s2, q2

    s2, q2 = lax.fori_loop(0, nb, p1, (zrow, zrow))

    a2, c2 = bn_affine(s2, q2, g2_ref, be2_ref)

    def p2(i, carry):
        hs = hblock(i) * a2 + c2
        out_t = lax.dot_general(W2_ref[...], hs, (((1,), (1,)), ((), ())),
                                preferred_element_type=jnp.float32,
                                precision=lax.Precision.HIGHEST)
        out_ref[i, :, :] = out_t + b2_ref[...]
        return carry

    lax.fori_loop(0, nb, p2, 0)


def kernel(causal, gamma1, beta1, W1, b1, gamma2, beta2, W2, b2):
    n, d = causal.shape
    d_out = W2.shape[0]
    blk = _pick_block(n)
    nb = n // blk

    row = lambda v: v.reshape(1, -1)
    vmem = pl.BlockSpec(memory_space=pltpu.MemorySpace.VMEM)

    fn = pl.pallas_call(
        functools.partial(_mlp_kernel, nb=nb, blk=blk, inv_n=1.0 / n),
        in_specs=[
            pl.BlockSpec(memory_space=pl.MemorySpace.ANY),
            vmem, vmem, vmem, vmem,
            vmem, vmem, vmem, vmem,
        ],
        out_specs=pl.BlockSpec(memory_space=pltpu.MemorySpace.VMEM),
        out_shape=jax.ShapeDtypeStruct((nb, d_out, blk), jnp.float32),
        scratch_shapes=[
            pltpu.VMEM((n, d), jnp.bfloat16),
            pltpu.VMEM((2, blk, d), jnp.float32),
            pltpu.SemaphoreType.DMA((2,)),
        ],
    )
    b2w = jnp.broadcast_to(b2.reshape(-1, 1), (d_out, blk))
    out3 = fn(causal, W1, row(b1), row(gamma1), row(beta1),
              W2, b2w, row(gamma2), row(beta2))
    return out3.transpose(0, 2, 1).reshape(n, d_out)


# speedup vs baseline: 2.8304x; 1.7430x over previous
import functools

import jax
import jax.numpy as jnp
from jax import lax
from jax.experimental import pallas as pl
from jax.experimental.pallas import tpu as pltpu

_EPS = 1e-5


def _pick_block(n):
    for blk in (4000, 2048, 2000, 1024, 1000, 512, 496, 256):
        if n % blk == 0:
            return blk
    return n


def _mlp_kernel(x_hbm, W1_ref, b1_ref, g1_ref, be1_ref, W2_ref,
                g2_ref, be2_ref, out_ref, br_ref, xc_ref, xbuf_ref, sem,
                *, nb, blk, inv_n):
    d = W1_ref.shape[0]

    def copy_in(slot, i):
        return pltpu.make_async_copy(
            x_hbm.at[pl.ds(i * blk, blk), :], xbuf_ref.at[slot],
            sem.at[slot])

    copy_in(0, 0).start()

    def p0(i, carry):
        s1, q1 = carry
        slot = lax.rem(i, 2)

        @pl.when(i + 1 < nb)
        def _prefetch():
            copy_in(lax.rem(i + 1, 2), i + 1).start()

        copy_in(slot, i).wait()
        xb = xbuf_ref[slot]
        s1 = s1 + jnp.sum(xb, axis=0, keepdims=True)
        q1 = q1 + jnp.sum(xb * xb, axis=0, keepdims=True)
        xc_ref[pl.ds(i * blk, blk), :] = xb.astype(jnp.bfloat16)
        return s1, q1

    zrow = jnp.zeros((1, d), jnp.float32)
    s1, q1 = lax.fori_loop(0, nb, p0, (zrow, zrow))

    def bn_affine(s, q, g_ref, be_ref):
        mean = s * inv_n
        var = q * inv_n - mean * mean
        a = g_ref[...] * lax.rsqrt(var + _EPS)
        c = be_ref[...] - mean * a
        return a, c

    a1, c1 = bn_affine(s1, q1, g1_ref, be1_ref)
    W1s = (W1_ref[...] * a1).astype(jnp.bfloat16)
    bias1 = lax.dot_general(c1, W1_ref[...], (((1,), (1,)), ((), ())),
                            preferred_element_type=jnp.float32) + b1_ref[...]

    def p1(i, carry):
        s2, q2 = carry
        xcb = xc_ref[pl.ds(i * blk, blk), :]
        z = lax.dot_general(xcb, W1s, (((1,), (1,)), ((), ())),
                            preferred_element_type=jnp.float32)
        h = jnp.maximum(z + bias1, 0.0)
        s2 = s2 + jnp.sum(h, axis=0, keepdims=True)
        q2 = q2 + jnp.sum(h * h, axis=0, keepdims=True)
        xc_ref[pl.ds(i * blk, blk), :] = h.astype(jnp.bfloat16)
        return s2, q2

    s2, q2 = lax.fori_loop(0, nb, p1, (zrow, zrow))

    a2, c2 = bn_affine(s2, q2, g2_ref, be2_ref)
    W2s = (W2_ref[...] * a2).astype(jnp.bfloat16)
    br_ref[...] = lax.dot_general(c2, W2_ref[...], (((1,), (1,)), ((), ())),
                                  preferred_element_type=jnp.float32)

    def p2(i, carry):
        hcb = xc_ref[pl.ds(i * blk, blk), :]
        out_t = lax.dot_general(W2s, hcb, (((1,), (1,)), ((), ())),
                                preferred_element_type=jnp.float32)
        out_ref[i, :, :] = out_t
        return carry

    lax.fori_loop(0, nb, p2, 0)


def kernel(causal, gamma1, beta1, W1, b1, gamma2, beta2, W2, b2):
    n, d = causal.shape
    d_out = W2.shape[0]
    blk = _pick_block(n)
    nb = n // blk

    row = lambda v: v.reshape(1, -1)
    vmem = pl.BlockSpec(memory_space=pltpu.MemorySpace.VMEM)

    fn = pl.pallas_call(
        functools.partial(_mlp_kernel, nb=nb, blk=blk, inv_n=1.0 / n),
        in_specs=[
            pl.BlockSpec(memory_space=pl.MemorySpace.ANY),
            vmem, vmem, vmem, vmem,
            vmem, vmem, vmem,
        ],
        out_specs=(vmem, vmem),
        out_shape=(jax.ShapeDtypeStruct((nb, d_out, blk), jnp.float32),
                   jax.ShapeDtypeStruct((1, d_out), jnp.float32)),
        scratch_shapes=[
            pltpu.VMEM((n, d), jnp.bfloat16),
            pltpu.VMEM((2, blk, d), jnp.float32),
            pltpu.SemaphoreType.DMA((2,)),
        ],
    )
    out3, brow = fn(causal, W1, row(b1), row(gamma1), row(beta1),
                    W2, row(gamma2), row(beta2))
    bias = brow + b2.reshape(1, -1)
    return out3.transpose(0, 2, 1).reshape(n, d_out) + bias


# speedup vs baseline: 3.3917x; 1.1983x over previous
import functools

import jax
import jax.numpy as jnp
from jax import lax
from jax.experimental import pallas as pl
from jax.experimental.pallas import tpu as pltpu

_EPS = 1e-5


def _pick_block(n):
    for blk in (10000, 4000, 2048, 2000, 1024, 1000, 512, 496, 256):
        if n % blk == 0:
            return blk
    return n


def _mlp_kernel(x_hbm, W1_ref, b1_ref, g1_ref, be1_ref, W2_ref,
                g2_ref, be2_ref, out_ref, br_ref, xc_ref, xbuf_ref, sem,
                *, nb, blk, inv_n):
    d = W1_ref.shape[0]

    def copy_in(slot, i):
        return pltpu.make_async_copy(
            x_hbm.at[pl.ds(i * blk, blk), :], xbuf_ref.at[slot],
            sem.at[slot])

    copy_in(0, 0).start()
    copy_in(1, 1).start()

    def p0(i, carry):
        s1, q1 = carry
        slot = lax.rem(i, 3)

        @pl.when(i + 2 < nb)
        def _prefetch():
            copy_in(lax.rem(i + 2, 3), i + 2).start()

        copy_in(slot, i).wait()
        xb = xbuf_ref[slot]
        s1 = s1 + jnp.sum(xb, axis=0, keepdims=True)
        q1 = q1 + jnp.sum(xb * xb, axis=0, keepdims=True)
        xc_ref[pl.ds(i * blk, blk), :] = xb.astype(jnp.bfloat16)
        return s1, q1

    zrow = jnp.zeros((1, d), jnp.float32)
    s1, q1 = lax.fori_loop(0, nb, p0, (zrow, zrow))

    def bn_affine(s, q, g_ref, be_ref):
        mean = s * inv_n
        var = q * inv_n - mean * mean
        a = g_ref[...] * lax.rsqrt(var + _EPS)
        c = be_ref[...] - mean * a
        return a, c

    a1, c1 = bn_affine(s1, q1, g1_ref, be1_ref)
    W1s = (W1_ref[...] * a1).astype(jnp.bfloat16)
    bias1 = lax.dot_general(c1, W1_ref[...], (((1,), (1,)), ((), ())),
                            preferred_element_type=jnp.float32) + b1_ref[...]

    def p1(i, carry):
        s2, q2 = carry
        xcb = xc_ref[pl.ds(i * blk, blk), :]
        z = lax.dot_general(xcb, W1s, (((1,), (1,)), ((), ())),
                            preferred_element_type=jnp.float32)
        h = jnp.maximum(z + bias1, 0.0)
        s2 = s2 + jnp.sum(h, axis=0, keepdims=True)
        q2 = q2 + jnp.sum(h * h, axis=0, keepdims=True)
        xc_ref[pl.ds(i * blk, blk), :] = h.astype(jnp.bfloat16)
        return s2, q2

    s2, q2 = lax.fori_loop(0, nb, p1, (zrow, zrow))

    a2, c2 = bn_affine(s2, q2, g2_ref, be2_ref)
    W2s = (W2_ref[...] * a2).astype(jnp.bfloat16)
    br_ref[...] = lax.dot_general(c2, W2_ref[...], (((1,), (1,)), ((), ())),
                                  preferred_element_type=jnp.float32)

    def p2(i, carry):
        hcb = xc_ref[pl.ds(i * blk, blk), :]
        out_t = lax.dot_general(W2s, hcb, (((1,), (1,)), ((), ())),
                                preferred_element_type=jnp.float32)
        out_ref[i, :, :] = out_t
        return carry

    lax.fori_loop(0, nb, p2, 0)


def kernel(causal, gamma1, beta1, W1, b1, gamma2, beta2, W2, b2):
    n, d = causal.shape
    d_out = W2.shape[0]
    blk = _pick_block(n)
    nb = n // blk

    row = lambda v: v.reshape(1, -1)
    vmem = pl.BlockSpec(memory_space=pltpu.MemorySpace.VMEM)

    fn = pl.pallas_call(
        functools.partial(_mlp_kernel, nb=nb, blk=blk, inv_n=1.0 / n),
        in_specs=[
            pl.BlockSpec(memory_space=pl.MemorySpace.ANY),
            vmem, vmem, vmem, vmem,
            vmem, vmem, vmem,
        ],
        out_specs=(vmem, vmem),
        out_shape=(jax.ShapeDtypeStruct((nb, d_out, blk), jnp.float32),
                   jax.ShapeDtypeStruct((1, d_out), jnp.float32)),
        scratch_shapes=[
            pltpu.VMEM((n, d), jnp.bfloat16),
            pltpu.VMEM((3, blk, d), jnp.float32),
            pltpu.SemaphoreType.DMA((3,)),
        ],
    )
    out3, brow = fn(causal, W1, row(b1), row(gamma1), row(beta1),
                    W2, row(gamma2), row(beta2))
    bias = brow + b2.reshape(1, -1)
    return out3.transpose(0, 2, 1).reshape(n, d_out) + bias
